# Initial kernel scaffold; baseline (speedup 1.0000x reference)
import jax
import jax.numpy as jnp
from jax.experimental import pallas as pl


def kernel(patches, embs, locations, Wq0, Wk0, Wv0, Wp0, We0, Wq1, Wk1, Wv1, Wp1, We1):
    raise NotImplementedError("write your pallas kernel here")


# speedup vs baseline: 6.4461x; 6.4461x over previous
import jax
import jax.numpy as jnp
from jax.experimental import pallas as pl


def _fusion_kernel(u_ref, e_ref, wq0_ref, wk0_ref, wv0_ref, wp0_ref, we0_ref,
                   wq1_ref, wk1_ref, wv1_ref, wp1_ref, o_ref):
    f32 = jnp.float32
    u = u_ref[0]
    e8 = e_ref[0]
    e = jnp.concatenate([e8, jnp.mean(e8, axis=0, keepdims=True)], axis=0)
    d = u.shape[-1]
    scale = jax.lax.rsqrt(f32(d))

    def mm(a, b):
        return jnp.dot(a, b, preferred_element_type=f32)

    def attend(q, k, v):
        lg = jax.lax.dot_general(q, k, (((1,), (1,)), ((), ())),
                                 preferred_element_type=f32) * scale
        lg = lg - jnp.max(lg, axis=-1, keepdims=True)
        w = jnp.exp(lg)
        a = w / jnp.sum(w, axis=-1, keepdims=True)
        return mm(a, v)

    inj0 = attend(mm(u, wq0_ref[...]), mm(e, wk0_ref[...]), mm(e, wv0_ref[...]))
    xm = mm(u, wp0_ref[...]) + inj0
    e1 = mm(e, we0_ref[...])
    inj1 = attend(mm(xm, wq1_ref[...]), mm(e1, wk1_ref[...]), mm(e1, wv1_ref[...]))
    o_ref[0] = mm(xm, wp1_ref[...]) + inj1


def kernel(patches, embs, locations, Wq0, Wk0, Wv0, Wp0, We0,
           Wq1, Wk1, Wv1, Wp1, We1):
    del locations, We1
    b, h, w, d0 = patches.shape
    p = h * w
    m0 = embs.shape[1]
    u = patches.reshape(b, p, d0)
    wspec = pl.BlockSpec((d0, d0), lambda i: (0, 0))
    return pl.pallas_call(
        _fusion_kernel,
        grid=(b,),
        in_specs=[pl.BlockSpec((1, p, d0), lambda i: (i, 0, 0)),
                  pl.BlockSpec((1, m0, d0), lambda i: (i, 0, 0))] + [wspec] * 9,
        out_specs=pl.BlockSpec((1, p, d0), lambda i: (i, 0, 0)),
        out_shape=jax.ShapeDtypeStruct((b, p, d0), jnp.float32),
    )(u, embs, Wq0, Wk0, Wv0, Wp0, We0, Wq1, Wk1, Wv1, Wp1)
